# baseline (speedup 1.0000x reference)
import jax
import jax.numpy as jnp
from jax import lax
from jax.experimental import pallas as pl
from jax.experimental.pallas import tpu as pltpu
from jax.experimental.pallas import tpu_sc as plsc

HIDDEN = 64
WIDE = 2 * HIDDEN
NC, NS = 2, 16
NW = NC * NS
W = 128


def kernel(input_ids, table):
    B, T = input_ids.shape
    n = B * T
    per_w = n // NW
    steps = per_w // W
    ids_flat = input_ids.reshape(n)
    tbl_wide = table.reshape(table.shape[0] // 2, WIDE)

    mesh = plsc.VectorSubcoreMesh(core_axis_name="c", subcore_axis_name="s")

    @jax.jit
    def run(tbl, ids):
        @pl.kernel(
            out_type=jax.ShapeDtypeStruct((n, HIDDEN), jnp.float32),
            mesh=mesh,
            scratch_types=[
                pltpu.VMEM((W,), jnp.int32),
                pltpu.VMEM((W,), jnp.int32),
                pltpu.VMEM((W,), jnp.int32),
                pltpu.VMEM((W,), jnp.int32),
                pltpu.VMEM((W, WIDE), jnp.float32),
                pltpu.VMEM((W, WIDE), jnp.float32),
                pltpu.VMEM((W, HIDDEN), jnp.float32),
                pltpu.VMEM((W, HIDDEN), jnp.float32),
                pltpu.SemaphoreType.DMA,
                pltpu.SemaphoreType.DMA,
                pltpu.SemaphoreType.DMA,
                pltpu.SemaphoreType.DMA,
                pltpu.SemaphoreType.DMA,
                pltpu.SemaphoreType.DMA,
            ],
        )
        def k(
            tbl_hbm,
            ids_hbm,
            out_hbm,
            idx0,
            idx1,
            gidx0,
            gidx1,
            g0,
            g1,
            c0,
            c1,
            semiv0,
            semiv1,
            semg0,
            semg1,
            semo0,
            semo1,
        ):
            idx = (idx0, idx1)
            gidx = (gidx0, gidx1)
            g = (g0, g1)
            c = (c0, c1)
            semiv = (semiv0, semiv1)
            semg = (semg0, semg1)
            semo = (semo0, semo1)

            wid = lax.axis_index("s") * NC + lax.axis_index("c")
            base = wid * per_w

            def start_idx(s, b):
                off = base + s * W
                pltpu.async_copy(ids_hbm.at[pl.ds(off, W)], idx[b], semiv[b])

            def wait_idx(b):
                pltpu.make_async_copy(
                    ids_hbm.at[pl.ds(0, W)], idx[b], semiv[b]
                ).wait()

            def compute_gidx(b):
                @pl.loop(0, W // 16)
                def _(j):
                    sl = pl.ds(j * 16, 16)
                    gidx[b][sl] = lax.shift_right_logical(idx[b][sl], 1)

            def start_gather(b):
                pltpu.async_copy(tbl_hbm.at[gidx[b]], g[b], semg[b])

            def wait_gather(b):
                pltpu.make_async_copy(tbl_hbm.at[gidx[b]], g[b], semg[b]).wait()

            def extract(b):
                @pl.loop(0, W // 16)
                def _(w16):
                    w0 = w16 * 16
                    pv = (idx[b][pl.ds(w0, 16)] & 1) * HIDDEN
                    for r in range(16):
                        off = pv[r]
                        w = w0 + r
                        for j in range(HIDDEN // 16):
                            c[b][w, pl.ds(j * 16, 16)] = g[b][
                                w, pl.ds(off + j * 16, 16)
                            ]

            def start_out(s, b):
                off = base + s * W
                pltpu.async_copy(c[b], out_hbm.at[pl.ds(off, W)], semo[b])

            def wait_out(b):
                pltpu.make_async_copy(
                    c[b], out_hbm.at[pl.ds(0, W)], semo[b]
                ).wait()

            start_idx(0, 0)
            start_idx(1, 1)
            wait_idx(0)
            compute_gidx(0)
            start_gather(0)

            def body(i, s, b):
                b1 = 1 - b
                @pl.when(s + 1 < steps)
                def _():
                    wait_idx(b1)
                    compute_gidx(b1)
                    start_gather(b1)

                wait_gather(b)

                @pl.when(s >= 2)
                def _():
                    wait_out(b)

                extract(b)
                start_out(s, b)

                @pl.when(s + 2 < steps)
                def _():
                    start_idx(s + 2, b)

            @pl.loop(0, steps // 2)
            def _(i):
                body(i, 2 * i, 0)
                body(i, 2 * i + 1, 1)

            wait_out(0)
            wait_out(1)

        return k(tbl, ids)

    return run(tbl_wide, ids_flat).reshape(B, T, HIDDEN)


# speedup vs baseline: 1.0318x; 1.0318x over previous
import jax
import jax.numpy as jnp
from jax import lax
from jax.experimental import pallas as pl
from jax.experimental.pallas import tpu as pltpu
from jax.experimental.pallas import tpu_sc as plsc

HIDDEN = 64
NC, NS = 2, 16
NW = NC * NS
W = 128
NBUF = 4


def kernel(input_ids, table):
    B, T = input_ids.shape
    n = B * T
    per_w = n // NW
    steps = per_w // W
    ids_flat = input_ids.reshape(n)

    mesh = plsc.VectorSubcoreMesh(core_axis_name="c", subcore_axis_name="s")

    @jax.jit
    def run(tbl, ids):
        @pl.kernel(
            out_type=jax.ShapeDtypeStruct((n, HIDDEN), jnp.float32),
            mesh=mesh,
            compiler_params=pltpu.CompilerParams(use_tc_tiling_on_sc=False),
            scratch_types=[
                pltpu.VMEM((W,), jnp.int32),
                pltpu.VMEM((W,), jnp.int32),
                pltpu.VMEM((W,), jnp.int32),
                pltpu.VMEM((W,), jnp.int32),
                pltpu.VMEM((W, HIDDEN), jnp.float32),
                pltpu.VMEM((W, HIDDEN), jnp.float32),
                pltpu.VMEM((W, HIDDEN), jnp.float32),
                pltpu.VMEM((W, HIDDEN), jnp.float32),
                pltpu.SemaphoreType.DMA,
                pltpu.SemaphoreType.DMA,
                pltpu.SemaphoreType.DMA,
                pltpu.SemaphoreType.DMA,
                pltpu.SemaphoreType.DMA,
                pltpu.SemaphoreType.DMA,
                pltpu.SemaphoreType.DMA,
                pltpu.SemaphoreType.DMA,
                pltpu.SemaphoreType.DMA,
                pltpu.SemaphoreType.DMA,
                pltpu.SemaphoreType.DMA,
                pltpu.SemaphoreType.DMA,
            ],
        )
        def k(
            tbl_hbm,
            ids_hbm,
            out_hbm,
            idx0,
            idx1,
            idx2,
            idx3,
            g0,
            g1,
            g2,
            g3,
            semi0,
            semi1,
            semi2,
            semi3,
            semg0,
            semg1,
            semg2,
            semg3,
            semo0,
            semo1,
            semo2,
            semo3,
        ):
            idx = (idx0, idx1, idx2, idx3)
            g = (g0, g1, g2, g3)
            semi = (semi0, semi1, semi2, semi3)
            semg = (semg0, semg1, semg2, semg3)
            semo = (semo0, semo1, semo2, semo3)

            wid = lax.axis_index("s") * NC + lax.axis_index("c")
            base = wid * per_w

            def start_idx(s, b):
                off = base + s * W
                pltpu.async_copy(ids_hbm.at[pl.ds(off, W)], idx[b], semi[b])

            def wait_idx(b):
                pltpu.make_async_copy(
                    ids_hbm.at[pl.ds(0, W)], idx[b], semi[b]
                ).wait()

            def start_gather(b):
                pltpu.async_copy(tbl_hbm.at[idx[b]], g[b], semg[b])

            def wait_gather(b):
                pltpu.make_async_copy(tbl_hbm.at[idx[b]], g[b], semg[b]).wait()

            def start_out(s, b):
                off = base + s * W
                pltpu.async_copy(g[b], out_hbm.at[pl.ds(off, W)], semo[b])

            def wait_out(b):
                pltpu.make_async_copy(
                    g[b], out_hbm.at[pl.ds(0, W)], semo[b]
                ).wait()

            start_idx(0, 0)
            start_idx(1, 1)
            start_idx(2, 2)
            start_idx(3, 3)
            wait_idx(0)
            start_gather(0)

            def body(s, b):
                b1 = (b + 1) % NBUF

                @pl.when(s + 1 < steps)
                def _():
                    wait_idx(b1)

                    @pl.when(s >= NBUF - 1)
                    def _():
                        wait_out(b1)

                    start_gather(b1)

                wait_gather(b)
                start_out(s, b)

                @pl.when(s + NBUF < steps)
                def _():
                    start_idx(s + NBUF, b)

            @pl.loop(0, steps // NBUF)
            def _(i):
                body(NBUF * i, 0)
                body(NBUF * i + 1, 1)
                body(NBUF * i + 2, 2)
                body(NBUF * i + 3, 3)

            wait_out(0)
            wait_out(1)
            wait_out(2)
            wait_out(3)

        return k(tbl, ids)

    return run(table, ids_flat).reshape(B, T, HIDDEN)


# speedup vs baseline: 1.0376x; 1.0056x over previous
import jax
import jax.numpy as jnp
from jax import lax
from jax.experimental import pallas as pl
from jax.experimental.pallas import tpu as pltpu
from jax.experimental.pallas import tpu_sc as plsc

HIDDEN = 64
NC, NS = 2, 16
NW = NC * NS
W = 256
NBUF = 4


def kernel(input_ids, table):
    B, T = input_ids.shape
    n = B * T
    per_w = n // NW
    steps = per_w // W
    ids_flat = input_ids.reshape(n)

    mesh = plsc.VectorSubcoreMesh(core_axis_name="c", subcore_axis_name="s")

    @jax.jit
    def run(tbl, ids):
        @pl.kernel(
            out_type=jax.ShapeDtypeStruct((n, HIDDEN), jnp.float32),
            mesh=mesh,
            compiler_params=pltpu.CompilerParams(use_tc_tiling_on_sc=False),
            scratch_types=[
                pltpu.VMEM((per_w,), jnp.int32),
                pltpu.VMEM((W, HIDDEN), jnp.float32),
                pltpu.VMEM((W, HIDDEN), jnp.float32),
                pltpu.VMEM((W, HIDDEN), jnp.float32),
                pltpu.VMEM((W, HIDDEN), jnp.float32),
                pltpu.SemaphoreType.DMA,
                pltpu.SemaphoreType.DMA,
                pltpu.SemaphoreType.DMA,
                pltpu.SemaphoreType.DMA,
                pltpu.SemaphoreType.DMA,
                pltpu.SemaphoreType.DMA,
                pltpu.SemaphoreType.DMA,
                pltpu.SemaphoreType.DMA,
                pltpu.SemaphoreType.DMA,
            ],
        )
        def k(
            tbl_hbm,
            ids_hbm,
            out_hbm,
            ids_all,
            g0,
            g1,
            g2,
            g3,
            semi,
            semg0,
            semg1,
            semg2,
            semg3,
            semo0,
            semo1,
            semo2,
            semo3,
        ):
            g = (g0, g1, g2, g3)
            semg = (semg0, semg1, semg2, semg3)
            semo = (semo0, semo1, semo2, semo3)

            wid = lax.axis_index("s") * NC + lax.axis_index("c")
            base = wid * per_w

            pltpu.async_copy(ids_hbm.at[pl.ds(base, per_w)], ids_all, semi)
            pltpu.make_async_copy(
                ids_hbm.at[pl.ds(0, per_w)], ids_all, semi
            ).wait()

            def start_gather(s, b):
                pltpu.async_copy(
                    tbl_hbm.at[ids_all.at[pl.ds(s * W, W)]], g[b], semg[b]
                )

            def wait_gather(b):
                pltpu.make_async_copy(
                    tbl_hbm.at[ids_all.at[pl.ds(0, W)]], g[b], semg[b]
                ).wait()

            def start_out(s, b):
                off = base + s * W
                pltpu.async_copy(g[b], out_hbm.at[pl.ds(off, W)], semo[b])

            def wait_out(b):
                pltpu.make_async_copy(
                    g[b], out_hbm.at[pl.ds(0, W)], semo[b]
                ).wait()

            start_gather(0, 0)
            start_gather(1, 1)
            start_gather(2, 2)

            def body(s, b):
                b3 = (b + 3) % NBUF

                wait_gather(b)
                start_out(s, b)

                @pl.when(s + 3 < steps)
                def _():
                    @pl.when(s >= 1)
                    def _():
                        wait_out(b3)

                    start_gather(s + 3, b3)

            @pl.loop(0, steps // NBUF)
            def _(i):
                body(NBUF * i, 0)
                body(NBUF * i + 1, 1)
                body(NBUF * i + 2, 2)
                body(NBUF * i + 3, 3)

            wait_out(0)
            wait_out(1)
            wait_out(2)
            wait_out(3)

        return k(tbl, ids)

    return run(table, ids_flat).reshape(B, T, HIDDEN)
